# Initial kernel scaffold; baseline (speedup 1.0000x reference)
import jax
import jax.numpy as jnp
from jax.experimental import pallas as pl


def kernel(x, edge_index, n_node_features, mini_batch, W1, a1, Wout, aout):
    raise NotImplementedError("write your pallas kernel here")


# speedup vs baseline: 1.8436x; 1.8436x over previous
import jax
import jax.numpy as jnp
from jax.experimental import pallas as pl
from jax.experimental.pallas import tpu as pltpu

N = 4096
NFEAT = 128
NHID = 16
NCLASS = 32
NHEADS = 8
ALPHA = 0.2
TELEPORT = 0.1

BLK = 256


def _lrelu(v):
    return jnp.where(v > 0, v, ALPHA * v)


def _elu(v):
    return jnp.where(v > 0, v, jnp.exp(v) - 1.0)


def _proj_kernel(x_ref, w_ref, a1_ref, a2_ref,
                 wh_ref, e1_ref, e2_ref, c_ref, mean_ref):
    wh = jnp.dot(x_ref[:], w_ref[:], preferred_element_type=jnp.float32)
    wh_ref[:] = wh
    e1 = jnp.dot(wh, a1_ref[:], preferred_element_type=jnp.float32)
    e2 = jnp.dot(wh, a2_ref[:], preferred_element_type=jnp.float32)
    e1_ref[:] = e1
    e2_ref[:] = e2
    m = jnp.max(e2, axis=0, keepdims=True)
    c_ref[:] = _lrelu(e1 + m)
    mean_ref[:] = jnp.mean(wh, axis=0, keepdims=True)


def _att1_kernel(adj_ref, e1_ref, c_ref, e2t_ref, wh_ref, mean_ref, out_ref):
    i = pl.program_id(0)
    adj = adj_ref[:]
    mask = adj > 0
    wh_rows = wh_ref[pl.ds(i * BLK, BLK), :]
    for h in range(NHEADS):
        e1c = e1_ref[:, h:h + 1]
        cc = c_ref[:, h:h + 1]
        s = _lrelu(e1c + e2t_ref[h:h + 1, :]) - cc
        w = jnp.where(mask, jnp.exp(s), 0.0)
        den = jnp.sum(w, axis=1, keepdims=True)
        num = jnp.dot(w, wh_ref[:, h * NHID:(h + 1) * NHID],
                      preferred_element_type=jnp.float32)
        agg = jnp.where(den > 0, num / jnp.where(den > 0, den, 1.0),
                        mean_ref[:, h * NHID:(h + 1) * NHID])
        hp = TELEPORT * agg + (1.0 - TELEPORT) * wh_rows[:, h * NHID:(h + 1) * NHID]
        out_ref[:, h * NHID:(h + 1) * NHID] = _elu(hp)


def _att2_kernel(adj_ref, e1_ref, c_ref, e2t_ref, who_ref, mean_ref, out_ref):
    i = pl.program_id(0)
    mask = adj_ref[:] > 0
    s = _lrelu(e1_ref[:] + e2t_ref[:]) - c_ref[:]
    w = jnp.where(mask, jnp.exp(s), 0.0)
    den = jnp.sum(w, axis=1, keepdims=True)
    num = jnp.dot(w, who_ref[:], preferred_element_type=jnp.float32)
    agg = jnp.where(den > 0, num / jnp.where(den > 0, den, 1.0), mean_ref[:])
    who_rows = who_ref[pl.ds(i * BLK, BLK), :]
    out_ref[:] = _elu(TELEPORT * agg + (1.0 - TELEPORT) * who_rows)


def _projections(x, wcat, a1m, a2m, nheads, d):
    n = x.shape[0]
    dtot = nheads * d
    return pl.pallas_call(
        _proj_kernel,
        out_shape=(
            jax.ShapeDtypeStruct((n, dtot), jnp.float32),
            jax.ShapeDtypeStruct((n, nheads), jnp.float32),
            jax.ShapeDtypeStruct((n, nheads), jnp.float32),
            jax.ShapeDtypeStruct((n, nheads), jnp.float32),
            jax.ShapeDtypeStruct((1, dtot), jnp.float32),
        ),
    )(x, wcat, a1m, a2m)


def _attention1(adj, e1, c, e2t, wh, mean):
    n = adj.shape[0]
    grid = (n // BLK,)
    return pl.pallas_call(
        _att1_kernel,
        grid=grid,
        in_specs=[
            pl.BlockSpec((BLK, n), lambda i: (i, 0)),
            pl.BlockSpec((BLK, NHEADS), lambda i: (i, 0)),
            pl.BlockSpec((BLK, NHEADS), lambda i: (i, 0)),
            pl.BlockSpec((NHEADS, n), lambda i: (0, 0)),
            pl.BlockSpec((n, NHEADS * NHID), lambda i: (0, 0)),
            pl.BlockSpec((1, NHEADS * NHID), lambda i: (0, 0)),
        ],
        out_specs=pl.BlockSpec((BLK, NHEADS * NHID), lambda i: (i, 0)),
        out_shape=jax.ShapeDtypeStruct((n, NHEADS * NHID), jnp.float32),
    )(adj, e1, c, e2t, wh, mean)


def _attention2(adj, e1, c, e2t, who, mean):
    n = adj.shape[0]
    grid = (n // BLK,)
    return pl.pallas_call(
        _att2_kernel,
        grid=grid,
        in_specs=[
            pl.BlockSpec((BLK, n), lambda i: (i, 0)),
            pl.BlockSpec((BLK, 1), lambda i: (i, 0)),
            pl.BlockSpec((BLK, 1), lambda i: (i, 0)),
            pl.BlockSpec((1, n), lambda i: (0, 0)),
            pl.BlockSpec((n, NCLASS), lambda i: (0, 0)),
            pl.BlockSpec((1, NCLASS), lambda i: (0, 0)),
        ],
        out_specs=pl.BlockSpec((BLK, NCLASS), lambda i: (i, 0)),
        out_shape=jax.ShapeDtypeStruct((n, NCLASS), jnp.float32),
    )(adj, e1, c, e2t, who, mean)


def kernel(x, edge_index, n_node_features, mini_batch, W1, a1, Wout, aout):
    x = x.astype(jnp.float32)
    adj = edge_index

    wcat = jnp.transpose(W1, (1, 0, 2)).reshape(NFEAT, NHEADS * NHID)
    eye = jnp.eye(NHEADS, dtype=jnp.float32)
    a1m = (eye[:, None, :] * a1[:, :NHID, 0][:, :, None]).reshape(
        NHEADS * NHID, NHEADS)
    a2m = (eye[:, None, :] * a1[:, NHID:, 0][:, :, None]).reshape(
        NHEADS * NHID, NHEADS)

    wh, e1, e2, c, mean = _projections(x, wcat, a1m, a2m, NHEADS, NHID)
    e2t = e2.T
    h = _attention1(adj, e1, c, e2t, wh, mean)

    who, e1o, e2o, co, meano = _projections(
        x=h, wcat=Wout.astype(jnp.float32),
        a1m=aout[:NCLASS].astype(jnp.float32),
        a2m=aout[NCLASS:].astype(jnp.float32), nheads=1, d=NCLASS)
    e2ot = e2o.reshape(1, N)
    out = _attention2(adj, e1o, co, e2ot, who, meano)
    return out


# speedup vs baseline: 2.7737x; 1.5045x over previous
import jax
import jax.numpy as jnp
from jax.experimental import pallas as pl
from jax.experimental.pallas import tpu as pltpu

N = 4096
NFEAT = 128
NHID = 16
NCLASS = 32
NHEADS = 8
ALPHA = 0.2
TELEPORT = 0.1
LOG2E = 1.4426950408889634

BLK = 256


def _lrelu(v):
    return jnp.maximum(v, ALPHA * v)


def _elu(v):
    return jnp.where(v > 0, v, jnp.exp(v) - 1.0)


def _proj_kernel(x_ref, w_ref, a1_ref, a2_ref, p_ref, ones_ref,
                 whaug_ref, e1f_ref, df_ref, e2_ref, mean_ref):
    wh = jnp.dot(x_ref[:], w_ref[:], preferred_element_type=jnp.float32)
    e1 = jnp.dot(wh, a1_ref[:], preferred_element_type=jnp.float32)
    e2 = jnp.dot(wh, a2_ref[:], preferred_element_type=jnp.float32)
    e2_ref[:] = e2
    m = jnp.max(e2, axis=0, keepdims=True)
    c = _lrelu(e1 + m)
    e1f_ref[:] = e1 - c
    df_ref[:] = -(1.0 - ALPHA) * c
    whaug_ref[:] = (jnp.dot(wh, p_ref[:], preferred_element_type=jnp.float32)
                    + ones_ref[:])
    mean_ref[:] = jnp.mean(wh, axis=0, keepdims=True)


def _att1_kernel(adj_ref, e1f_ref, df_ref, e2t_ref, whaug_ref, mean_ref,
                 out_ref):
    i = pl.program_id(0)
    maskf = (adj_ref[:] > 0).astype(jnp.float32)
    wha_rows = whaug_ref[pl.ds(i * BLK, BLK), :]
    for h in range(NHEADS):
        s = 2 * NHID * h
        t = e1f_ref[:, h:h + 1] + e2t_ref[h:h + 1, :]
        u = jnp.maximum(t, ALPHA * t + df_ref[:, h:h + 1])
        w = jnp.exp2(u) * maskf
        nd = jnp.dot(w, whaug_ref[:, s:s + 2 * NHID],
                     preferred_element_type=jnp.float32)
        num = nd[:, :NHID]
        den = nd[:, NHID:NHID + 1]
        ok = den > 0
        rec = 1.0 / jnp.where(ok, den, 1.0)
        agg = jnp.where(ok, num * rec, mean_ref[:, NHID * h:NHID * (h + 1)])
        hp = (TELEPORT * agg
              + (1.0 - TELEPORT) * wha_rows[:, s:s + NHID])
        out_ref[:, NHID * h:NHID * (h + 1)] = _elu(hp)


def _att2_kernel(adj_ref, e1f_ref, df_ref, e2t_ref, whaug_ref, mean_ref,
                 out_ref):
    i = pl.program_id(0)
    maskf = (adj_ref[:] > 0).astype(jnp.float32)
    t = e1f_ref[:] + e2t_ref[:]
    u = jnp.maximum(t, ALPHA * t + df_ref[:])
    w = jnp.exp2(u) * maskf
    nd = jnp.dot(w, whaug_ref[:], preferred_element_type=jnp.float32)
    num = nd[:, :NCLASS]
    den = nd[:, NCLASS:NCLASS + 1]
    ok = den > 0
    rec = 1.0 / jnp.where(ok, den, 1.0)
    agg = jnp.where(ok, num * rec, mean_ref[:])
    who_rows = whaug_ref[pl.ds(i * BLK, BLK), :NCLASS]
    out_ref[:] = _elu(TELEPORT * agg + (1.0 - TELEPORT) * who_rows)


def _projections(x, wcat, a1m, a2m, pmat, onesb, nheads, d):
    n = x.shape[0]
    dtot = nheads * d
    return pl.pallas_call(
        _proj_kernel,
        out_shape=(
            jax.ShapeDtypeStruct((n, 2 * dtot), jnp.float32),
            jax.ShapeDtypeStruct((n, nheads), jnp.float32),
            jax.ShapeDtypeStruct((n, nheads), jnp.float32),
            jax.ShapeDtypeStruct((n, nheads), jnp.float32),
            jax.ShapeDtypeStruct((1, dtot), jnp.float32),
        ),
    )(x, wcat, a1m, a2m, pmat, onesb)


def _attention1(adj, e1f, df, e2t, whaug, mean):
    n = adj.shape[0]
    return pl.pallas_call(
        _att1_kernel,
        grid=(n // BLK,),
        in_specs=[
            pl.BlockSpec((BLK, n), lambda i: (i, 0)),
            pl.BlockSpec((BLK, NHEADS), lambda i: (i, 0)),
            pl.BlockSpec((BLK, NHEADS), lambda i: (i, 0)),
            pl.BlockSpec((NHEADS, n), lambda i: (0, 0)),
            pl.BlockSpec((n, 2 * NHEADS * NHID), lambda i: (0, 0)),
            pl.BlockSpec((1, NHEADS * NHID), lambda i: (0, 0)),
        ],
        out_specs=pl.BlockSpec((BLK, NHEADS * NHID), lambda i: (i, 0)),
        out_shape=jax.ShapeDtypeStruct((n, NHEADS * NHID), jnp.float32),
    )(adj, e1f, df, e2t, whaug, mean)


def _attention2(adj, e1f, df, e2t, whaug, mean):
    n = adj.shape[0]
    return pl.pallas_call(
        _att2_kernel,
        grid=(n // BLK,),
        in_specs=[
            pl.BlockSpec((BLK, n), lambda i: (i, 0)),
            pl.BlockSpec((BLK, 1), lambda i: (i, 0)),
            pl.BlockSpec((BLK, 1), lambda i: (i, 0)),
            pl.BlockSpec((1, n), lambda i: (0, 0)),
            pl.BlockSpec((n, 2 * NCLASS), lambda i: (0, 0)),
            pl.BlockSpec((1, NCLASS), lambda i: (0, 0)),
        ],
        out_specs=pl.BlockSpec((BLK, NCLASS), lambda i: (i, 0)),
        out_shape=jax.ShapeDtypeStruct((n, NCLASS), jnp.float32),
    )(adj, e1f, df, e2t, whaug, mean)


def _placement(nheads, d):
    dtot = nheads * d
    p = jnp.zeros((dtot, 2 * dtot), jnp.float32)
    rows = jnp.arange(dtot)
    cols = (rows // d) * 2 * d + (rows % d)
    p = p.at[rows, cols].set(1.0)
    ones = jnp.zeros((1, 2 * dtot), jnp.float32)
    ones = ones.at[0, (jnp.arange(nheads) * 2 * d) + d].set(1.0)
    return p, ones


def kernel(x, edge_index, n_node_features, mini_batch, W1, a1, Wout, aout):
    x = x.astype(jnp.float32)
    adj = edge_index

    wcat = jnp.transpose(W1, (1, 0, 2)).reshape(NFEAT, NHEADS * NHID)
    eye = jnp.eye(NHEADS, dtype=jnp.float32)
    a1m = LOG2E * (eye[:, None, :] * a1[:, :NHID, 0][:, :, None]).reshape(
        NHEADS * NHID, NHEADS)
    a2m = LOG2E * (eye[:, None, :] * a1[:, NHID:, 0][:, :, None]).reshape(
        NHEADS * NHID, NHEADS)
    p1, ones1 = _placement(NHEADS, NHID)

    whaug, e1f, df, e2, mean = _projections(x, wcat, a1m, a2m, p1, ones1,
                                            NHEADS, NHID)
    h = _attention1(adj, e1f, df, e2.T, whaug, mean)

    p2, ones2 = _placement(1, NCLASS)
    whaug2, e1f2, df2, e22, mean2 = _projections(
        h, Wout.astype(jnp.float32),
        LOG2E * aout[:NCLASS].astype(jnp.float32),
        LOG2E * aout[NCLASS:].astype(jnp.float32),
        p2, ones2, 1, NCLASS)
    out = _attention2(adj, e1f2, df2, e22.reshape(1, N), whaug2, mean2)
    return out


# speedup vs baseline: 3.2020x; 1.1544x over previous
import jax
import jax.numpy as jnp
from jax.experimental import pallas as pl
from jax.experimental.pallas import tpu as pltpu

N = 4096
NFEAT = 128
NHID = 16
NCLASS = 32
NHEADS = 8
ALPHA = 0.2
TELEPORT = 0.1
LOG2E = 1.4426950408889634

BLK = 256


def _lrelu(v):
    return jnp.maximum(v, ALPHA * v)


def _elu(v):
    return jnp.where(v > 0, v, jnp.exp(v) - 1.0)


def _proj_kernel(x_ref, w_ref, a1_ref, a2_ref, p_ref, ones_ref,
                 wh_ref, whaug_ref, e1f_ref, df_ref, e2_ref, mean_ref):
    wh = jnp.dot(x_ref[:], w_ref[:], preferred_element_type=jnp.float32)
    wh_ref[:] = wh
    e1 = jnp.dot(wh, a1_ref[:], preferred_element_type=jnp.float32)
    e2 = jnp.dot(wh, a2_ref[:], preferred_element_type=jnp.float32)
    e2_ref[:] = e2.astype(jnp.bfloat16)
    m = jnp.max(e2, axis=0, keepdims=True)
    c = _lrelu(e1 + m)
    e1f_ref[:] = (e1 - c).astype(jnp.bfloat16)
    df_ref[:] = (-(1.0 - ALPHA) * c).astype(jnp.bfloat16)
    whaug_ref[:] = (jnp.dot(wh, p_ref[:], preferred_element_type=jnp.float32)
                    + ones_ref[:]).astype(jnp.bfloat16)
    mean_ref[:] = jnp.mean(wh, axis=0, keepdims=True)


def _att1_kernel(adj_ref, e1f_ref, df_ref, e2t_ref, wh_ref, whaug_ref,
                 mean_ref, out_ref):
    i = pl.program_id(0)
    maskf = (adj_ref[:] > 0).astype(jnp.bfloat16)
    wh_rows = wh_ref[pl.ds(i * BLK, BLK), :]
    alpha = jnp.bfloat16(ALPHA)
    for h in range(NHEADS):
        s = 2 * NHID * h
        t = e1f_ref[:, h:h + 1] + e2t_ref[h:h + 1, :]
        u = jnp.maximum(t, alpha * t + df_ref[:, h:h + 1])
        w = jnp.exp2(u) * maskf
        nd = jnp.dot(w, whaug_ref[:, s:s + 2 * NHID],
                     preferred_element_type=jnp.float32)
        num = nd[:, :NHID]
        den = nd[:, NHID:NHID + 1]
        ok = den > 0
        rec = 1.0 / jnp.where(ok, den, 1.0)
        agg = jnp.where(ok, num * rec, mean_ref[:, NHID * h:NHID * (h + 1)])
        hp = (TELEPORT * agg
              + (1.0 - TELEPORT) * wh_rows[:, NHID * h:NHID * (h + 1)])
        out_ref[:, NHID * h:NHID * (h + 1)] = _elu(hp)


def _att2_kernel(adj_ref, e1f_ref, df_ref, e2t_ref, wh_ref, whaug_ref,
                 mean_ref, out_ref):
    i = pl.program_id(0)
    maskf = (adj_ref[:] > 0).astype(jnp.bfloat16)
    t = e1f_ref[:] + e2t_ref[:]
    u = jnp.maximum(t, jnp.bfloat16(ALPHA) * t + df_ref[:])
    w = jnp.exp2(u) * maskf
    nd = jnp.dot(w, whaug_ref[:], preferred_element_type=jnp.float32)
    num = nd[:, :NCLASS]
    den = nd[:, NCLASS:NCLASS + 1]
    ok = den > 0
    rec = 1.0 / jnp.where(ok, den, 1.0)
    agg = jnp.where(ok, num * rec, mean_ref[:])
    who_rows = wh_ref[pl.ds(i * BLK, BLK), :]
    out_ref[:] = _elu(TELEPORT * agg + (1.0 - TELEPORT) * who_rows)


def _projections(x, wcat, a1m, a2m, pmat, onesb, nheads, d):
    n = x.shape[0]
    dtot = nheads * d
    return pl.pallas_call(
        _proj_kernel,
        out_shape=(
            jax.ShapeDtypeStruct((n, dtot), jnp.float32),
            jax.ShapeDtypeStruct((n, 2 * dtot), jnp.bfloat16),
            jax.ShapeDtypeStruct((n, nheads), jnp.bfloat16),
            jax.ShapeDtypeStruct((n, nheads), jnp.bfloat16),
            jax.ShapeDtypeStruct((n, nheads), jnp.bfloat16),
            jax.ShapeDtypeStruct((1, dtot), jnp.float32),
        ),
    )(x, wcat, a1m, a2m, pmat, onesb)


def _attention1(adj, e1f, df, e2t, wh, whaug, mean):
    n = adj.shape[0]
    return pl.pallas_call(
        _att1_kernel,
        grid=(n // BLK,),
        in_specs=[
            pl.BlockSpec((BLK, n), lambda i: (i, 0)),
            pl.BlockSpec((BLK, NHEADS), lambda i: (i, 0)),
            pl.BlockSpec((BLK, NHEADS), lambda i: (i, 0)),
            pl.BlockSpec((NHEADS, n), lambda i: (0, 0)),
            pl.BlockSpec((n, NHEADS * NHID), lambda i: (0, 0)),
            pl.BlockSpec((n, 2 * NHEADS * NHID), lambda i: (0, 0)),
            pl.BlockSpec((1, NHEADS * NHID), lambda i: (0, 0)),
        ],
        out_specs=pl.BlockSpec((BLK, NHEADS * NHID), lambda i: (i, 0)),
        out_shape=jax.ShapeDtypeStruct((n, NHEADS * NHID), jnp.float32),
    )(adj, e1f, df, e2t, wh, whaug, mean)


def _attention2(adj, e1f, df, e2t, wh, whaug, mean):
    n = adj.shape[0]
    return pl.pallas_call(
        _att2_kernel,
        grid=(n // BLK,),
        in_specs=[
            pl.BlockSpec((BLK, n), lambda i: (i, 0)),
            pl.BlockSpec((BLK, 1), lambda i: (i, 0)),
            pl.BlockSpec((BLK, 1), lambda i: (i, 0)),
            pl.BlockSpec((1, n), lambda i: (0, 0)),
            pl.BlockSpec((n, NCLASS), lambda i: (0, 0)),
            pl.BlockSpec((n, 2 * NCLASS), lambda i: (0, 0)),
            pl.BlockSpec((1, NCLASS), lambda i: (0, 0)),
        ],
        out_specs=pl.BlockSpec((BLK, NCLASS), lambda i: (i, 0)),
        out_shape=jax.ShapeDtypeStruct((n, NCLASS), jnp.float32),
    )(adj, e1f, df, e2t, wh, whaug, mean)


def _placement(nheads, d):
    dtot = nheads * d
    p = jnp.zeros((dtot, 2 * dtot), jnp.float32)
    rows = jnp.arange(dtot)
    cols = (rows // d) * 2 * d + (rows % d)
    p = p.at[rows, cols].set(1.0)
    ones = jnp.zeros((1, 2 * dtot), jnp.float32)
    ones = ones.at[0, (jnp.arange(nheads) * 2 * d) + d].set(1.0)
    return p, ones


def kernel(x, edge_index, n_node_features, mini_batch, W1, a1, Wout, aout):
    x = x.astype(jnp.float32)
    adj = edge_index

    wcat = jnp.transpose(W1, (1, 0, 2)).reshape(NFEAT, NHEADS * NHID)
    eye = jnp.eye(NHEADS, dtype=jnp.float32)
    a1m = LOG2E * (eye[:, None, :] * a1[:, :NHID, 0][:, :, None]).reshape(
        NHEADS * NHID, NHEADS)
    a2m = LOG2E * (eye[:, None, :] * a1[:, NHID:, 0][:, :, None]).reshape(
        NHEADS * NHID, NHEADS)
    p1, ones1 = _placement(NHEADS, NHID)

    wh, whaug, e1f, df, e2, mean = _projections(x, wcat, a1m, a2m, p1, ones1,
                                                NHEADS, NHID)
    h = _attention1(adj, e1f, df, e2.T, wh, whaug, mean)

    p2, ones2 = _placement(1, NCLASS)
    wh2, whaug2, e1f2, df2, e22, mean2 = _projections(
        h, Wout.astype(jnp.float32),
        LOG2E * aout[:NCLASS].astype(jnp.float32),
        LOG2E * aout[NCLASS:].astype(jnp.float32),
        p2, ones2, 1, NCLASS)
    out = _attention2(adj, e1f2, df2, e22.reshape(1, N), wh2, whaug2, mean2)
    return out
